# baseline (device time: 101446 ns/iter reference)
import jax
import jax.numpy as jnp
from jax import lax
from jax.experimental import pallas as pl
from jax.experimental.pallas import tpu as pltpu


def kernel(x, pi):
    def body(x_ref, pi_ref, out_ref, send_sem, recv_sem):
        my_x = lax.axis_index("x")
        my_y = lax.axis_index("y")
        my_z = lax.axis_index("z")
        tgt_y = pi_ref[my_y]

        barrier_sem = pltpu.get_barrier_semaphore()

        @pl.when(tgt_y == my_y)
        def _():
            out_ref[...] = x_ref[...]

        @pl.when(tgt_y != my_y)
        def _():
            pl.semaphore_signal(
                barrier_sem,
                inc=1,
                device_id=(my_x, tgt_y, my_z),
                device_id_type=pl.DeviceIdType.MESH,
            )
            pl.semaphore_wait(barrier_sem, 1)

            rdma = pltpu.make_async_remote_copy(
                src_ref=x_ref,
                dst_ref=out_ref,
                send_sem=send_sem,
                recv_sem=recv_sem,
                device_id=(my_x, tgt_y, my_z),
                device_id_type=pl.DeviceIdType.MESH,
            )
            rdma.start()
            rdma.wait()

    return pl.pallas_call(
        body,
        out_shape=jax.ShapeDtypeStruct(x.shape, x.dtype),
        in_specs=[
            pl.BlockSpec(memory_space=pltpu.VMEM),
            pl.BlockSpec(memory_space=pltpu.SMEM),
        ],
        out_specs=pl.BlockSpec(memory_space=pltpu.VMEM),
        scratch_shapes=[
            pltpu.SemaphoreType.DMA,
            pltpu.SemaphoreType.DMA,
        ],
        compiler_params=pltpu.CompilerParams(collective_id=0),
    )(x, pi)


# device time: 49335 ns/iter; 2.0563x vs baseline; 2.0563x over previous
import jax
import jax.numpy as jnp
from jax import lax
from jax.experimental import pallas as pl
from jax.experimental.pallas import tpu as pltpu

CHUNK = 64
QROWS = 512
NQC = QROWS // CHUNK
Y_DIAG = (0, 1, 2)
A_DIAG = (3, 4, 5)
B_DIAG = (6, 7)
N_Y = NQC + len(Y_DIAG)
N_CW = NQC + len(A_DIAG)
N_CCW = NQC + len(B_DIAG)


def kernel(x, pi):
    def body(
        x_ref,
        pi_ref,
        out_ref,
        y_snd,
        y_rcv,
        cw_snd,
        cw_rcv,
        ccw_snd,
        ccw_rcv,
        dummy_sem,
    ):
        my_x = lax.axis_index("x")
        my_y = lax.axis_index("y")
        my_z = lax.axis_index("z")
        tgt_y = pi_ref[my_y]

        barrier_sem = pltpu.get_barrier_semaphore()

        @pl.when(tgt_y == my_y)
        def _():
            out_ref[...] = x_ref[...]

        @pl.when(tgt_y != my_y)
        def _():
            p = 2 * my_z + (my_x ^ my_z)

            def ring_coords(q):
                zq = q // 2
                xq = zq ^ (q % 2)
                return (xq, my_y, zq)

            p_next = (p + 1) % 4
            p_prev = (p + 3) % 4
            p_diag = (p + 2) % 4
            dev_next = ring_coords(p_next)
            dev_prev = ring_coords(p_prev)
            dev_y = (my_x, tgt_y, my_z)
            dev_me = (my_x, my_y, my_z)

            q_me = p * QROWS
            q_next = p_next * QROWS
            q_prev = p_prev * QROWS
            q_diag = p_diag * QROWS

            def rows(ref, base, c):
                return ref.at[:, pl.ds(base + c * CHUNK, CHUNK), :]

            send_descs = []

            def send(src, dst, snd_sem, rcv_sem, dev):
                r = pltpu.make_async_remote_copy(
                    src_ref=src,
                    dst_ref=dst,
                    send_sem=snd_sem,
                    recv_sem=rcv_sem,
                    device_id=dev,
                    device_id_type=pl.DeviceIdType.MESH,
                )
                r.start()
                send_descs.append(r)

            def wait_recv(dst, rcv_sem):
                pltpu.make_async_remote_copy(
                    src_ref=dst,
                    dst_ref=dst,
                    send_sem=dummy_sem,
                    recv_sem=rcv_sem,
                    device_id=dev_me,
                    device_id_type=pl.DeviceIdType.MESH,
                ).wait_recv()

            for dev in (dev_y, dev_next, dev_prev):
                pl.semaphore_signal(
                    barrier_sem,
                    inc=1,
                    device_id=dev,
                    device_id_type=pl.DeviceIdType.MESH,
                )
            pl.semaphore_wait(barrier_sem, 3)

            for c in range(NQC):
                send(
                    rows(x_ref, q_me, c),
                    rows(out_ref, q_me, c),
                    y_snd.at[c],
                    y_rcv.at[c],
                    dev_y,
                )
            for i, c in enumerate(Y_DIAG):
                send(
                    rows(x_ref, q_diag, c),
                    rows(out_ref, q_diag, c),
                    y_snd.at[NQC + i],
                    y_rcv.at[NQC + i],
                    dev_y,
                )

            for c in range(NQC):
                wait_recv(rows(out_ref, q_me, c), y_rcv.at[c])
                send(
                    rows(out_ref, q_me, c),
                    rows(out_ref, q_me, c),
                    cw_snd.at[c],
                    cw_rcv.at[c],
                    dev_next,
                )
                send(
                    rows(out_ref, q_me, c),
                    rows(out_ref, q_me, c),
                    ccw_snd.at[c],
                    ccw_rcv.at[c],
                    dev_prev,
                )

            for i, c in enumerate(A_DIAG):
                wait_recv(rows(out_ref, q_prev, c), cw_rcv.at[c])
                send(
                    rows(out_ref, q_prev, c),
                    rows(out_ref, q_prev, c),
                    cw_snd.at[NQC + i],
                    cw_rcv.at[NQC + i],
                    dev_next,
                )

            for i, c in enumerate(B_DIAG):
                wait_recv(rows(out_ref, q_next, c), ccw_rcv.at[c])
                send(
                    rows(out_ref, q_next, c),
                    rows(out_ref, q_next, c),
                    ccw_snd.at[NQC + i],
                    ccw_rcv.at[NQC + i],
                    dev_prev,
                )

            for i, c in enumerate(Y_DIAG):
                wait_recv(rows(out_ref, q_diag, c), y_rcv.at[NQC + i])
            for c in range(NQC):
                if c not in A_DIAG:
                    wait_recv(rows(out_ref, q_prev, c), cw_rcv.at[c])
            for i, c in enumerate(A_DIAG):
                wait_recv(rows(out_ref, q_diag, c), cw_rcv.at[NQC + i])
            for c in range(NQC):
                if c not in B_DIAG:
                    wait_recv(rows(out_ref, q_next, c), ccw_rcv.at[c])
            for i, c in enumerate(B_DIAG):
                wait_recv(rows(out_ref, q_diag, c), ccw_rcv.at[NQC + i])

            for r in send_descs:
                r.wait_send()

    return pl.pallas_call(
        body,
        out_shape=jax.ShapeDtypeStruct(x.shape, x.dtype),
        in_specs=[
            pl.BlockSpec(memory_space=pltpu.VMEM),
            pl.BlockSpec(memory_space=pltpu.SMEM),
        ],
        out_specs=pl.BlockSpec(memory_space=pltpu.VMEM),
        scratch_shapes=[
            pltpu.SemaphoreType.DMA((N_Y,)),
            pltpu.SemaphoreType.DMA((N_Y,)),
            pltpu.SemaphoreType.DMA((N_CW,)),
            pltpu.SemaphoreType.DMA((N_CW,)),
            pltpu.SemaphoreType.DMA((N_CCW,)),
            pltpu.SemaphoreType.DMA((N_CCW,)),
            pltpu.SemaphoreType.DMA,
        ],
        compiler_params=pltpu.CompilerParams(collective_id=0),
    )(x, pi)


# device time: 47148 ns/iter; 2.1517x vs baseline; 1.0464x over previous
import jax
import jax.numpy as jnp
from jax import lax
from jax.experimental import pallas as pl
from jax.experimental.pallas import tpu as pltpu

CHUNK = 32
QROWS = 512
NQC = QROWS // CHUNK
Y_DIAG = (0, 1, 2, 3, 4, 5)
A_DIAG = (6, 7, 8, 9, 10)
B_DIAG = (11, 12, 13, 14, 15)
N_Y = NQC + len(Y_DIAG)
N_CW = NQC + len(A_DIAG)
N_CCW = NQC + len(B_DIAG)


def kernel(x, pi):
    def body(
        x_ref,
        pi_ref,
        out_ref,
        y_snd,
        y_rcv,
        cw_snd,
        cw_rcv,
        ccw_snd,
        ccw_rcv,
        dummy_sem,
    ):
        my_x = lax.axis_index("x")
        my_y = lax.axis_index("y")
        my_z = lax.axis_index("z")
        tgt_y = pi_ref[my_y]

        barrier_sem = pltpu.get_barrier_semaphore()

        @pl.when(tgt_y == my_y)
        def _():
            out_ref[...] = x_ref[...]

        @pl.when(tgt_y != my_y)
        def _():
            p = 2 * my_z + (my_x ^ my_z)

            def ring_coords(q):
                zq = q // 2
                xq = zq ^ (q % 2)
                return (xq, my_y, zq)

            p_next = (p + 1) % 4
            p_prev = (p + 3) % 4
            p_diag = (p + 2) % 4
            dev_next = ring_coords(p_next)
            dev_prev = ring_coords(p_prev)
            dev_y = (my_x, tgt_y, my_z)
            dev_me = (my_x, my_y, my_z)

            q_me = p * QROWS
            q_next = p_next * QROWS
            q_prev = p_prev * QROWS
            q_diag = p_diag * QROWS

            def rows(ref, base, c):
                return ref.at[:, pl.ds(base + c * CHUNK, CHUNK), :]

            send_descs = []

            def send(src, dst, snd_sem, rcv_sem, dev):
                r = pltpu.make_async_remote_copy(
                    src_ref=src,
                    dst_ref=dst,
                    send_sem=snd_sem,
                    recv_sem=rcv_sem,
                    device_id=dev,
                    device_id_type=pl.DeviceIdType.MESH,
                )
                r.start()
                send_descs.append(r)

            def wait_recv(dst, rcv_sem):
                pltpu.make_async_remote_copy(
                    src_ref=dst,
                    dst_ref=dst,
                    send_sem=dummy_sem,
                    recv_sem=rcv_sem,
                    device_id=dev_me,
                    device_id_type=pl.DeviceIdType.MESH,
                ).wait_recv()

            for dev in (dev_y, dev_next, dev_prev):
                pl.semaphore_signal(
                    barrier_sem,
                    inc=1,
                    device_id=dev,
                    device_id_type=pl.DeviceIdType.MESH,
                )
            pl.semaphore_wait(barrier_sem, 3)

            for c in range(NQC):
                send(
                    rows(x_ref, q_me, c),
                    rows(out_ref, q_me, c),
                    y_snd.at[c],
                    y_rcv.at[c],
                    dev_y,
                )
            for i, c in enumerate(Y_DIAG):
                send(
                    rows(x_ref, q_diag, c),
                    rows(out_ref, q_diag, c),
                    y_snd.at[NQC + i],
                    y_rcv.at[NQC + i],
                    dev_y,
                )

            for c in range(NQC):
                wait_recv(rows(out_ref, q_me, c), y_rcv.at[c])
                send(
                    rows(out_ref, q_me, c),
                    rows(out_ref, q_me, c),
                    cw_snd.at[c],
                    cw_rcv.at[c],
                    dev_next,
                )
                send(
                    rows(out_ref, q_me, c),
                    rows(out_ref, q_me, c),
                    ccw_snd.at[c],
                    ccw_rcv.at[c],
                    dev_prev,
                )

            for i, c in enumerate(A_DIAG):
                wait_recv(rows(out_ref, q_prev, c), cw_rcv.at[c])
                send(
                    rows(out_ref, q_prev, c),
                    rows(out_ref, q_prev, c),
                    cw_snd.at[NQC + i],
                    cw_rcv.at[NQC + i],
                    dev_next,
                )

            for i, c in enumerate(B_DIAG):
                wait_recv(rows(out_ref, q_next, c), ccw_rcv.at[c])
                send(
                    rows(out_ref, q_next, c),
                    rows(out_ref, q_next, c),
                    ccw_snd.at[NQC + i],
                    ccw_rcv.at[NQC + i],
                    dev_prev,
                )

            for i, c in enumerate(Y_DIAG):
                wait_recv(rows(out_ref, q_diag, c), y_rcv.at[NQC + i])
            for c in range(NQC):
                if c not in A_DIAG:
                    wait_recv(rows(out_ref, q_prev, c), cw_rcv.at[c])
            for i, c in enumerate(A_DIAG):
                wait_recv(rows(out_ref, q_diag, c), cw_rcv.at[NQC + i])
            for c in range(NQC):
                if c not in B_DIAG:
                    wait_recv(rows(out_ref, q_next, c), ccw_rcv.at[c])
            for i, c in enumerate(B_DIAG):
                wait_recv(rows(out_ref, q_diag, c), ccw_rcv.at[NQC + i])

            for r in send_descs:
                r.wait_send()

    return pl.pallas_call(
        body,
        out_shape=jax.ShapeDtypeStruct(x.shape, x.dtype),
        in_specs=[
            pl.BlockSpec(memory_space=pltpu.VMEM),
            pl.BlockSpec(memory_space=pltpu.SMEM),
        ],
        out_specs=pl.BlockSpec(memory_space=pltpu.VMEM),
        scratch_shapes=[
            pltpu.SemaphoreType.DMA((N_Y,)),
            pltpu.SemaphoreType.DMA((N_Y,)),
            pltpu.SemaphoreType.DMA((N_CW,)),
            pltpu.SemaphoreType.DMA((N_CW,)),
            pltpu.SemaphoreType.DMA((N_CCW,)),
            pltpu.SemaphoreType.DMA((N_CCW,)),
            pltpu.SemaphoreType.DMA,
        ],
        compiler_params=pltpu.CompilerParams(collective_id=0),
    )(x, pi)
